# baseline (speedup 1.0000x reference)
import jax
import jax.numpy as jnp
from jax.experimental import pallas as pl
from jax.experimental.pallas import tpu as pltpu

_B, _C, _H, _W = 8, 96, 224, 224
_E = 64
_CB = 32
_NJ = (_C + _E) // _CB


def _body(idx_ref, emb_seg_ref, img_ref, out_ref):
    j = pl.program_id(1)

    @pl.when(j < _E // _CB)
    def _emb():
        seg = emb_seg_ref[0, 0, 0, :]
        out_ref[0] = jnp.broadcast_to(seg[:, None, None], (_CB, _H, _W))

    @pl.when(j >= _E // _CB)
    def _img():
        out_ref[0] = img_ref[0]


def kernel(class_idx, image, emb_table):
    ne = _E // _CB
    return pl.pallas_call(
        _body,
        grid_spec=pltpu.PrefetchScalarGridSpec(
            num_scalar_prefetch=1,
            grid=(_B, _NJ),
            in_specs=[
                pl.BlockSpec(
                    (1, 1, 1, _CB),
                    lambda b, j, idx_ref: (idx_ref[b], jnp.minimum(j, ne - 1), 0, 0),
                ),
                pl.BlockSpec(
                    (1, _CB, _H, _W),
                    lambda b, j, idx_ref: (b, jnp.maximum(j - ne, 0), 0, 0),
                ),
            ],
            out_specs=pl.BlockSpec((1, _CB, _H, _W),
                                   lambda b, j, idx_ref: (b, j, 0, 0)),
        ),
        out_shape=jax.ShapeDtypeStruct((_B, _C + _E, _H, _W), jnp.float32),
    )(class_idx, emb_table.reshape(-1, ne, 1, _CB), image)
